# baseline (speedup 1.0000x reference)
import functools

import jax
import jax.numpy as jnp
from jax import lax
from jax.experimental import pallas as pl
from jax.experimental.pallas import tpu as pltpu
from jax.experimental.pallas import tpu_sc as plsc

_L = 16
_ROWS_PER_CHUNK = 2


def _rsqrt_newton(x):
    i = lax.bitcast_convert_type(x, jnp.int32)
    i = jnp.int32(0x5F3759DF) - lax.shift_right_logical(i, 1)
    y = lax.bitcast_convert_type(i, jnp.float32)
    for _ in range(2):
        y = y * (1.5 - 0.5 * x * y * y)
    return y


_GATHER_DNUMS = lax.GatherDimensionNumbers(
    offset_dims=(), collapsed_slice_dims=(0,), start_index_map=(0,))


def _lane_permute(v, idx):
    return lax.gather(v, idx[:, None], _GATHER_DNUMS, slice_sizes=(1,),
                      mode=lax.GatherScatterMode.PROMISE_IN_BOUNDS)


def _lane_sum(v, perm_idx):
    for idx in perm_idx:
        v = v + _lane_permute(v, idx)
    return v


def _make_sc_kernel(num_slots, seq_len, parts, dim, max_pos, num_workers):
    batch = num_slots // seq_len
    slots_per_chunk = _ROWS_PER_CHUNK * seq_len
    chunks = batch // _ROWS_PER_CHUNK // num_workers
    assert chunks % 2 == 0
    ncg = dim // _L
    row_ids = seq_len * parts
    out_per_chunk = slots_per_chunk * dim

    mesh = plsc.VectorSubcoreMesh(core_axis_name="c", subcore_axis_name="s")

    @functools.partial(
        pl.kernel,
        out_type=jax.ShapeDtypeStruct((num_slots * dim,), jnp.float32),
        mesh=mesh,
        scratch_types=[
            pltpu.VMEM((2, _ROWS_PER_CHUNK, row_ids), jnp.int32),
            pltpu.VMEM((2, _ROWS_PER_CHUNK, row_ids, dim), jnp.float32),
            pltpu.VMEM((2, out_per_chunk), jnp.float32),
            pltpu.VMEM((max_pos, dim), jnp.float32),
            pltpu.VMEM((dim,), jnp.float32),
            pltpu.VMEM((dim,), jnp.float32),
            pltpu.SemaphoreType.DMA,
            pltpu.SemaphoreType.DMA,
            pltpu.SemaphoreType.DMA,
            pltpu.SemaphoreType.DMA,
            pltpu.SemaphoreType.DMA,
        ],
        compiler_params=pltpu.CompilerParams(use_tc_tiling_on_sc=False),
    )
    def body(ids_ref, table_ref, pos_ref, gam_ref, bet_ref, out_ref,
             idx_v, rows_v, out_v, pos_v, gam_v, bet_v,
             gsem0, gsem1, isem0, isem1, osem):
        nc = mesh.num_cores
        wid = lax.axis_index("s") * nc + lax.axis_index("c")
        wchunk = wid * chunks
        gsems = (gsem0, gsem1)
        isems = (isem0, isem1)

        pltpu.sync_copy(pos_ref, pos_v)
        pltpu.sync_copy(gam_ref, gam_v)
        pltpu.sync_copy(bet_ref, bet_v)
        gam = [gam_v[pl.ds(cg * _L, _L)] for cg in range(ncg)]
        bet = [bet_v[pl.ds(cg * _L, _L)] for cg in range(ncg)]
        inv_parts = jnp.float32(1.0 / parts)
        inv_dim = jnp.float32(1.0 / dim)
        lanes = lax.iota(jnp.int32, _L)
        perm_idx = [lanes ^ jnp.int32(sh) for sh in (8, 4, 2, 1)]

        def ids_src(c):
            return ids_ref.at[pl.ds((wchunk + c) * _ROWS_PER_CHUNK,
                                    _ROWS_PER_CHUNK)]

        def gather_descs(b, sem):
            return [
                pltpu.make_async_copy(
                    table_ref.at[idx_v.at[b, jj]],
                    rows_v.at[b, jj],
                    sem)
                for jj in range(_ROWS_PER_CHUNK)
            ]

        def compute(b):
            @plsc.parallel_loop(0, slots_per_chunk, unroll=8)
            def _(i):
                r = lax.div(i, seq_len)
                p = lax.rem(i, seq_len)
                base = p * parts
                obase = i * dim
                accs = []
                for cg in range(ncg):
                    col = cg * _L
                    a = rows_v[b, r, base, pl.ds(col, _L)]
                    for k in range(1, parts):
                        a = a + rows_v[b, r, base + k, pl.ds(col, _L)]
                    a = a * inv_parts + pos_v[p, pl.ds(col, _L)]
                    accs.append(a)
                t = (accs[0] + accs[1]) + (accs[2] + accs[3])
                sq = (accs[0] * accs[0] + accs[1] * accs[1]) + (
                    accs[2] * accs[2] + accs[3] * accs[3])
                mean = _lane_sum(t, perm_idx) * inv_dim
                var = _lane_sum(sq, perm_idx) * inv_dim - mean * mean
                inv = _rsqrt_newton(var + jnp.float32(1e-5))
                for cg in range(ncg):
                    out_v[b, pl.ds(obase + cg * _L, _L)] = (
                        (accs[cg] - mean) * inv * gam[cg] + bet[cg])

        def out_dst(c):
            return out_ref.at[pl.ds((wchunk + c) * out_per_chunk,
                                    out_per_chunk)]

        pltpu.sync_copy(ids_src(0), idx_v.at[0])
        for d in gather_descs(0, gsems[0]):
            d.start()
        pltpu.async_copy(ids_src(1), idx_v.at[1], isems[1])

        def pair_body(c0, _):
            for b in range(2):
                c = c0 + b
                nb = 1 - b
                @pl.when(c + 1 < chunks)
                def _():
                    pltpu.make_async_copy(
                        ids_src(c + 1), idx_v.at[nb], isems[nb]).wait()
                    for d in gather_descs(nb, gsems[nb]):
                        d.start()
                for d in gather_descs(b, gsems[b]):
                    d.wait()

                @pl.when(c + 2 < chunks)
                def _():
                    pltpu.async_copy(ids_src(c + 2), idx_v.at[b], isems[b])
                compute(b)

                @pl.when(c > 0)
                def _():
                    pltpu.make_async_copy(
                        out_v.at[nb], out_dst(c - 1), osem).wait()
                pltpu.async_copy(out_v.at[b], out_dst(c), osem)
            return 0

        lax.fori_loop(0, chunks // 2, lambda j, x: pair_body(j * 2, x), 0)
        pltpu.make_async_copy(out_v.at[1], out_dst(chunks - 1), osem).wait()

    return body


def kernel(part_ids, part_table, pos_table, ln_gamma, ln_beta):
    batch, seq_len, parts = part_ids.shape
    dim = part_table.shape[1]
    num_slots = batch * seq_len
    ids2d = part_ids.reshape(batch, seq_len * parts)
    sc = _make_sc_kernel(num_slots, seq_len, parts, dim,
                         pos_table.shape[0], 32)
    out = sc(ids2d, part_table, pos_table, ln_gamma, ln_beta)
    return out.reshape(batch, seq_len, dim)


# speedup vs baseline: 1.3701x; 1.3701x over previous
import functools

import jax
import jax.numpy as jnp
from jax import lax
from jax.experimental import pallas as pl
from jax.experimental.pallas import tpu as pltpu
from jax.experimental.pallas import tpu_sc as plsc

_L = 16
_ROWS_PER_CHUNK = 2


def _rsqrt_newton(x):
    i = lax.bitcast_convert_type(x, jnp.int32)
    i = jnp.int32(0x5F3759DF) - lax.shift_right_logical(i, 1)
    y = lax.bitcast_convert_type(i, jnp.float32)
    for _ in range(2):
        y = y * (1.5 - 0.5 * x * y * y)
    return y


_GATHER_DNUMS = lax.GatherDimensionNumbers(
    offset_dims=(), collapsed_slice_dims=(0,), start_index_map=(0,))


def _lane_permute(v, idx):
    return lax.gather(v, idx[:, None], _GATHER_DNUMS, slice_sizes=(1,),
                      mode=lax.GatherScatterMode.PROMISE_IN_BOUNDS)


def _lane_sum(v, perm_idx):
    for idx in perm_idx:
        v = v + _lane_permute(v, idx)
    return v


def _make_sc_kernel(num_slots, seq_len, parts, dim, max_pos, num_workers):
    batch = num_slots // seq_len
    slots_per_chunk = _ROWS_PER_CHUNK * seq_len
    chunks = batch // _ROWS_PER_CHUNK // num_workers
    assert chunks % 2 == 0
    ncg = dim // _L
    row_ids = seq_len * parts
    out_per_chunk = slots_per_chunk * dim

    mesh = plsc.VectorSubcoreMesh(core_axis_name="c", subcore_axis_name="s")

    @functools.partial(
        pl.kernel,
        out_type=jax.ShapeDtypeStruct((num_slots * dim,), jnp.float32),
        mesh=mesh,
        scratch_types=[
            pltpu.VMEM((2, _ROWS_PER_CHUNK, row_ids), jnp.int32),
            pltpu.VMEM((2, _ROWS_PER_CHUNK, row_ids, dim), jnp.float32),
            pltpu.VMEM((2, out_per_chunk), jnp.float32),
            pltpu.VMEM((max_pos, dim), jnp.float32),
            pltpu.VMEM((dim,), jnp.float32),
            pltpu.VMEM((dim,), jnp.float32),
            pltpu.SemaphoreType.DMA,
            pltpu.SemaphoreType.DMA,
            pltpu.SemaphoreType.DMA,
            pltpu.SemaphoreType.DMA,
            pltpu.SemaphoreType.DMA,
        ],
        compiler_params=pltpu.CompilerParams(use_tc_tiling_on_sc=False),
    )
    def body(ids_ref, table_ref, pos_ref, gam_ref, bet_ref, out_ref,
             idx_v, rows_v, out_v, pos_v, gam_v, bet_v,
             gsem0, gsem1, isem0, isem1, osem):
        nc = mesh.num_cores
        wid = lax.axis_index("s") * nc + lax.axis_index("c")
        wchunk = wid * chunks
        gsems = (gsem0, gsem1)
        isems = (isem0, isem1)

        pltpu.sync_copy(pos_ref, pos_v)
        pltpu.sync_copy(gam_ref, gam_v)
        pltpu.sync_copy(bet_ref, bet_v)
        gam = [gam_v[pl.ds(cg * _L, _L)] for cg in range(ncg)]
        bet = [bet_v[pl.ds(cg * _L, _L)] for cg in range(ncg)]
        inv_parts = jnp.float32(1.0 / parts)
        inv_dim = jnp.float32(1.0 / dim)
        lanes = lax.iota(jnp.int32, _L)
        perm_idx = [lanes ^ jnp.int32(sh) for sh in (8, 4, 2, 1)]

        def ids_src(c):
            return ids_ref.at[pl.ds((wchunk + c) * _ROWS_PER_CHUNK,
                                    _ROWS_PER_CHUNK)]

        def gather_descs(b, sem):
            return [
                pltpu.make_async_copy(
                    table_ref.at[idx_v.at[b, jj]],
                    rows_v.at[b, jj],
                    sem)
                for jj in range(_ROWS_PER_CHUNK)
            ]

        def compute(b):
            @plsc.parallel_loop(0, slots_per_chunk, unroll=4)
            def _(i):
                r = lax.div(i, seq_len)
                p = lax.rem(i, seq_len)
                base = p * parts
                obase = i * dim
                accs = []
                for cg in range(ncg):
                    col = cg * _L
                    a = rows_v[b, r, base, pl.ds(col, _L)]
                    for k in range(1, parts):
                        a = a + rows_v[b, r, base + k, pl.ds(col, _L)]
                    a = a * inv_parts + pos_v[p, pl.ds(col, _L)]
                    accs.append(a)
                t = (accs[0] + accs[1]) + (accs[2] + accs[3])
                sq = (accs[0] * accs[0] + accs[1] * accs[1]) + (
                    accs[2] * accs[2] + accs[3] * accs[3])
                mean = _lane_sum(t, perm_idx) * inv_dim
                var = _lane_sum(sq, perm_idx) * inv_dim - mean * mean
                inv = _rsqrt_newton(var + jnp.float32(1e-5))
                for cg in range(ncg):
                    out_v[b, pl.ds(obase + cg * _L, _L)] = (
                        (accs[cg] - mean) * inv * gam[cg] + bet[cg])

        def out_dst(c):
            return out_ref.at[pl.ds((wchunk + c) * out_per_chunk,
                                    out_per_chunk)]

        pltpu.sync_copy(ids_src(0), idx_v.at[0])
        for d in gather_descs(0, gsems[0]):
            d.start()
        pltpu.async_copy(ids_src(1), idx_v.at[1], isems[1])

        def pair_body(c0, _):
            for b in range(2):
                c = c0 + b
                nb = 1 - b
                @pl.when(c + 1 < chunks)
                def _():
                    pltpu.make_async_copy(
                        ids_src(c + 1), idx_v.at[nb], isems[nb]).wait()
                    for d in gather_descs(nb, gsems[nb]):
                        d.start()
                for d in gather_descs(b, gsems[b]):
                    d.wait()

                @pl.when(c + 2 < chunks)
                def _():
                    pltpu.async_copy(ids_src(c + 2), idx_v.at[b], isems[b])
                compute(b)

                @pl.when(c > 0)
                def _():
                    pltpu.make_async_copy(
                        out_v.at[nb], out_dst(c - 1), osem).wait()
                pltpu.async_copy(out_v.at[b], out_dst(c), osem)
            return 0

        lax.fori_loop(0, chunks // 2, lambda j, x: pair_body(j * 2, x), 0)
        pltpu.make_async_copy(out_v.at[1], out_dst(chunks - 1), osem).wait()

    return body


def kernel(part_ids, part_table, pos_table, ln_gamma, ln_beta):
    batch, seq_len, parts = part_ids.shape
    dim = part_table.shape[1]
    num_slots = batch * seq_len
    ids2d = part_ids.reshape(batch, seq_len * parts)
    sc = _make_sc_kernel(num_slots, seq_len, parts, dim,
                         pos_table.shape[0], 32)
    out = sc(ids2d, part_table, pos_table, ln_gamma, ln_beta)
    return out.reshape(batch, seq_len, dim)
